# baseline (speedup 1.0000x reference)
import functools

import jax
import jax.numpy as jnp
from jax import lax
from jax.experimental import pallas as pl
from jax.experimental.pallas import tpu as pltpu
from jax.experimental.pallas import tpu_sc as plsc

_L = 16
_NSLOT = 3


@functools.lru_cache(maxsize=None)
def _build_sc_bel(B: int, W: int, H: int):
    NW = 32
    WC = 32
    units = (B * W) // WC
    assert units * WC == B * W and W % WC == 0
    upb = W // WC
    HG = H // _L
    assert HG * _L == H
    MAXK = (units + NW - 1) // NW
    full_k = units - (MAXK - 1) * NW

    mesh = plsc.VectorSubcoreMesh(core_axis_name="c", subcore_axis_name="s")

    data_bufs = [pltpu.VMEM((WC, H), jnp.float32) for _ in range(2 * _NSLOT)]
    const_bufs = [pltpu.VMEM((WC, H), jnp.float32) for _ in range(2)]
    sems = [pltpu.SemaphoreType.DMA for _ in range(2 * _NSLOT)]

    @functools.partial(
        pl.kernel,
        mesh=mesh,
        out_type=jax.ShapeDtypeStruct((B, W, 4, H), jnp.float32),
        scratch_types=data_bufs + const_bufs + sems,
        compiler_params=pltpu.CompilerParams(needs_layout_passes=False),
    )
    def body(in_hbm, bel_hbm, *sc):
        bufs = [sc[2 * s:2 * s + 2] for s in range(_NSLOT)]
        zb, ob = sc[2 * _NSLOT], sc[2 * _NSLOT + 1]
        sin = sc[2 * _NSLOT + 2:2 * _NSLOT + 2 + _NSLOT]
        sout = sc[2 * _NSLOT + 2 + _NSLOT:]
        wid = lax.axis_index("s") * 2 + lax.axis_index("c")
        zero_v = jnp.zeros((_L,), jnp.float32)
        one_v = jnp.ones((_L,), jnp.float32)

        def const_fill(w, c):
            def hbody(hg, c2):
                hs = pl.ds(hg * _L, _L)
                zb[w, hs] = zero_v
                ob[w, hs] = one_v
                return c2

            lax.fori_loop(0, HG, hbody, 0)
            return c

        lax.fori_loop(0, WC, const_fill, 0)

        def unit_pos(k):
            u = k * NW + wid
            return u // upb, pl.ds((u % upb) * WC, WC)

        def in_copies(k, s):
            b, ws = unit_pos(k)
            x0s, x1s = bufs[s]
            sem = sin[s]
            return [(in_hbm.at[b, 0, ws], x0s, sem),
                    (in_hbm.at[b, 1, ws], x1s, sem)]

        def out_copies(k, s):
            b, ws = unit_pos(k)
            x0s, x1s = bufs[s]
            sem = sout[s]
            return [(zb, bel_hbm.at[b, ws, 0], sem),
                    (x0s, bel_hbm.at[b, ws, 1], sem),
                    (x1s, bel_hbm.at[b, ws, 2], sem),
                    (ob, bel_hbm.at[b, ws, 3], sem)]

        def issue(copies):
            for src, dst, sem in copies:
                pltpu.async_copy(src, dst, sem)

        def drain(copies):
            for src, dst, sem in copies:
                pltpu.make_async_copy(src, dst, sem).wait()

        def stage_body(k, s):
            def body_fn():
                drain(in_copies(k, s))
                issue(out_copies(k, s))
            return body_fn

        def stage(k, s, first=False):
            if not first:
                drain(out_copies(k - 2, (s - 2) % _NSLOT))
            issue(in_copies(k + 1, (s + 1) % _NSLOT))
            stage_body(k, s)()

        issue(in_copies(0, 0))
        stage(0, 0, first=True)
        stage(1, 1, first=True)
        nbody = (MAXK - 4) // _NSLOT
        ktail = 2 + _NSLOT * nbody

        def group(g, c):
            k0 = _NSLOT * g + 2
            for j in range(_NSLOT):
                stage(k0 + j, (2 + j) % _NSLOT)
            return c

        lax.fori_loop(0, nbody, group, 0)
        for k in range(ktail, MAXK):
            s = k % _NSLOT
            if k == MAXK - 1 and full_k != NW:
                drain(out_copies(k - 2, (s - 2) % _NSLOT))
                pl.when(wid < full_k)(stage_body(k, s))
            else:
                nxt = k + 1
                if nxt < MAXK and (nxt < MAXK - 1 or full_k == NW):
                    stage(k, s)
                else:
                    drain(out_copies(k - 2, (s - 2) % _NSLOT))
                    if nxt < MAXK and nxt == MAXK - 1 and full_k != NW:
                        def issue_next(nxt=nxt, s=s):
                            issue(in_copies(nxt, (s + 1) % _NSLOT))
                        pl.when(wid < full_k)(issue_next)
                    stage_body(k, s)()
        drain(out_copies(MAXK - 2, (MAXK - 2) % _NSLOT))

        def drain_last():
            drain(out_copies(MAXK - 1, (MAXK - 1) % _NSLOT))

        if full_k == NW:
            drain_last()
        else:
            pl.when(wid < full_k)(drain_last)

    return body


@functools.lru_cache(maxsize=None)
def _build_tc_pl(B: int, W: int, H: int):
    TW = 96
    assert W % TW == 0

    def body(x_ref, o_ref):
        x0 = x_ref[0, 0]
        x1 = x_ref[0, 1]
        x2 = x_ref[0, 2]
        o_ref[0, :, 0, :] = jnp.zeros((TW, H), jnp.float32)
        o_ref[0, :, 1, :] = x0 + x2
        o_ref[0, :, 2, :] = x1 + x2
        o_ref[0, :, 3, :] = jnp.ones((TW, H), jnp.float32)

    return pl.pallas_call(
        body,
        grid=(B, W // TW),
        in_specs=[pl.BlockSpec((1, 3, TW, H), lambda b, w: (b, 0, w, 0))],
        out_specs=pl.BlockSpec((1, TW, 4, H), lambda b, w: (b, w, 0, 0)),
        out_shape=jax.ShapeDtypeStruct((B, W, 4, H), jnp.float32),
    )


def kernel(inputs):
    B, H, W, C = inputs.shape
    assert C == 3, "kernel specialized for a 2-class frame (3 input channels)"
    xt = jnp.transpose(inputs, (0, 3, 2, 1))
    bel_t = _build_sc_bel(B, W, H)(xt)
    pl_t = _build_tc_pl(B, W, H)(xt)
    bel = jnp.transpose(bel_t, (0, 3, 1, 2))
    pl_full = jnp.transpose(pl_t, (0, 3, 1, 2))
    return (bel, pl_full)


# speedup vs baseline: 1.2671x; 1.2671x over previous
import functools

import jax
import jax.numpy as jnp
from jax import lax
from jax.experimental import pallas as pl
from jax.experimental.pallas import tpu as pltpu
from jax.experimental.pallas import tpu_sc as plsc

_L = 16
_NSLOT = 3


@functools.lru_cache(maxsize=None)
def _build_sc_bel(B: int, W: int, H: int):
    NW = 32
    WC = 32
    units = (B * W) // WC
    assert units * WC == B * W and W % WC == 0
    upb = W // WC
    HG = H // _L
    assert HG * _L == H
    MAXK = (units + NW - 1) // NW
    full_k = units - (MAXK - 1) * NW

    mesh = plsc.VectorSubcoreMesh(core_axis_name="c", subcore_axis_name="s")

    data_bufs = [pltpu.VMEM((WC, H), jnp.float32) for _ in range(2 * _NSLOT)]
    const_bufs = [pltpu.VMEM((WC, H), jnp.float32) for _ in range(2)]
    sems = [pltpu.SemaphoreType.DMA for _ in range(2 * _NSLOT)]

    @functools.partial(
        pl.kernel,
        mesh=mesh,
        out_type=jax.ShapeDtypeStruct((B, W, 4, H), jnp.float32),
        scratch_types=data_bufs + const_bufs + sems,
        compiler_params=pltpu.CompilerParams(needs_layout_passes=False),
    )
    def body(in_hbm, bel_hbm, *sc):
        bufs = [sc[2 * s:2 * s + 2] for s in range(_NSLOT)]
        zb, ob = sc[2 * _NSLOT], sc[2 * _NSLOT + 1]
        sin = sc[2 * _NSLOT + 2:2 * _NSLOT + 2 + _NSLOT]
        sout = sc[2 * _NSLOT + 2 + _NSLOT:]
        wid = lax.axis_index("s") * 2 + lax.axis_index("c")
        zero_v = jnp.zeros((_L,), jnp.float32)
        one_v = jnp.ones((_L,), jnp.float32)

        def const_fill(w, c):
            def hbody(hg, c2):
                hs = pl.ds(hg * _L, _L)
                zb[w, hs] = zero_v
                ob[w, hs] = one_v
                return c2

            lax.fori_loop(0, HG, hbody, 0)
            return c

        lax.fori_loop(0, WC, const_fill, 0)

        def unit_pos(k):
            u = k * NW + wid
            return u // upb, pl.ds((u % upb) * WC, WC)

        def in_copies(k, s):
            b, ws = unit_pos(k)
            x0s, x1s = bufs[s]
            sem = sin[s]
            return [(in_hbm.at[b, 0, ws], x0s, sem),
                    (in_hbm.at[b, 1, ws], x1s, sem)]

        def out_copies(k, s):
            b, ws = unit_pos(k)
            x0s, x1s = bufs[s]
            sem = sout[s]
            return [(zb, bel_hbm.at[b, ws, 0], sem),
                    (x0s, bel_hbm.at[b, ws, 1], sem),
                    (x1s, bel_hbm.at[b, ws, 2], sem),
                    (ob, bel_hbm.at[b, ws, 3], sem)]

        def issue(copies):
            for src, dst, sem in copies:
                pltpu.async_copy(src, dst, sem)

        def drain(copies):
            for src, dst, sem in copies:
                pltpu.make_async_copy(src, dst, sem).wait()

        def stage_body(k, s):
            def body_fn():
                drain(in_copies(k, s))
                issue(out_copies(k, s))
            return body_fn

        def stage(k, s, first=False):
            if not first:
                drain(out_copies(k - 2, (s - 2) % _NSLOT))
            issue(in_copies(k + 1, (s + 1) % _NSLOT))
            stage_body(k, s)()

        issue(in_copies(0, 0))
        stage(0, 0, first=True)
        stage(1, 1, first=True)
        nbody = (MAXK - 4) // _NSLOT
        ktail = 2 + _NSLOT * nbody

        def group(g, c):
            k0 = _NSLOT * g + 2
            for j in range(_NSLOT):
                stage(k0 + j, (2 + j) % _NSLOT)
            return c

        lax.fori_loop(0, nbody, group, 0)
        for k in range(ktail, MAXK):
            s = k % _NSLOT
            if k == MAXK - 1 and full_k != NW:
                drain(out_copies(k - 2, (s - 2) % _NSLOT))
                pl.when(wid < full_k)(stage_body(k, s))
            else:
                nxt = k + 1
                if nxt < MAXK and (nxt < MAXK - 1 or full_k == NW):
                    stage(k, s)
                else:
                    drain(out_copies(k - 2, (s - 2) % _NSLOT))
                    if nxt < MAXK and nxt == MAXK - 1 and full_k != NW:
                        def issue_next(nxt=nxt, s=s):
                            issue(in_copies(nxt, (s + 1) % _NSLOT))
                        pl.when(wid < full_k)(issue_next)
                    stage_body(k, s)()
        drain(out_copies(MAXK - 2, (MAXK - 2) % _NSLOT))

        def drain_last():
            drain(out_copies(MAXK - 1, (MAXK - 1) % _NSLOT))

        if full_k == NW:
            drain_last()
        else:
            pl.when(wid < full_k)(drain_last)

    return body


@functools.lru_cache(maxsize=None)
def _build_tc_pl(B: int, W: int, H: int):
    TW = 312
    assert W % TW == 0

    def body(x_ref, o_ref):
        x0 = x_ref[0, 0]
        x1 = x_ref[0, 1]
        x2 = x_ref[0, 2]
        o_ref[0, :, 0, :] = jnp.zeros((TW, H), jnp.float32)
        o_ref[0, :, 1, :] = x0 + x2
        o_ref[0, :, 2, :] = x1 + x2
        o_ref[0, :, 3, :] = jnp.ones((TW, H), jnp.float32)

    return pl.pallas_call(
        body,
        grid=(B, W // TW),
        in_specs=[pl.BlockSpec((1, 3, TW, H), lambda b, w: (b, 0, w, 0))],
        out_specs=pl.BlockSpec((1, TW, 4, H), lambda b, w: (b, w, 0, 0)),
        out_shape=jax.ShapeDtypeStruct((B, W, 4, H), jnp.float32),
    )


def kernel(inputs):
    B, H, W, C = inputs.shape
    assert C == 3, "kernel specialized for a 2-class frame (3 input channels)"
    xt = jnp.transpose(inputs, (0, 3, 2, 1))
    bel_t = _build_sc_bel(B, W, H)(xt)
    pl_t = _build_tc_pl(B, W, H)(xt)
    bel = jnp.transpose(bel_t, (0, 3, 1, 2))
    pl_full = jnp.transpose(pl_t, (0, 3, 1, 2))
    return (bel, pl_full)


# speedup vs baseline: 1.2757x; 1.0068x over previous
import functools

import jax
import jax.numpy as jnp
from jax import lax
from jax.experimental import pallas as pl
from jax.experimental.pallas import tpu as pltpu
from jax.experimental.pallas import tpu_sc as plsc

_L = 16
_NSLOT = 3


@functools.lru_cache(maxsize=None)
def _build_sc_bel(B: int, W: int, H: int):
    NW = 32
    WC = 16
    units = (B * W) // WC
    assert units * WC == B * W and W % WC == 0
    upb = W // WC
    HG = H // _L
    assert HG * _L == H
    MAXK = (units + NW - 1) // NW
    full_k = units - (MAXK - 1) * NW

    mesh = plsc.VectorSubcoreMesh(core_axis_name="c", subcore_axis_name="s")

    data_bufs = [pltpu.VMEM((WC, H), jnp.float32) for _ in range(2 * _NSLOT)]
    const_bufs = [pltpu.VMEM((WC, H), jnp.float32) for _ in range(2)]
    sems = [pltpu.SemaphoreType.DMA for _ in range(2 * _NSLOT)]

    @functools.partial(
        pl.kernel,
        mesh=mesh,
        out_type=jax.ShapeDtypeStruct((B, W, 4, H), jnp.float32),
        scratch_types=data_bufs + const_bufs + sems,
        compiler_params=pltpu.CompilerParams(needs_layout_passes=False),
    )
    def body(in_hbm, bel_hbm, *sc):
        bufs = [sc[2 * s:2 * s + 2] for s in range(_NSLOT)]
        zb, ob = sc[2 * _NSLOT], sc[2 * _NSLOT + 1]
        sin = sc[2 * _NSLOT + 2:2 * _NSLOT + 2 + _NSLOT]
        sout = sc[2 * _NSLOT + 2 + _NSLOT:]
        wid = lax.axis_index("s") * 2 + lax.axis_index("c")
        zero_v = jnp.zeros((_L,), jnp.float32)
        one_v = jnp.ones((_L,), jnp.float32)

        def const_fill(w, c):
            def hbody(hg, c2):
                hs = pl.ds(hg * _L, _L)
                zb[w, hs] = zero_v
                ob[w, hs] = one_v
                return c2

            lax.fori_loop(0, HG, hbody, 0)
            return c

        lax.fori_loop(0, WC, const_fill, 0)

        def unit_pos(k):
            u = k * NW + wid
            return u // upb, pl.ds((u % upb) * WC, WC)

        def in_copies(k, s):
            b, ws = unit_pos(k)
            x0s, x1s = bufs[s]
            sem = sin[s]
            return [(in_hbm.at[b, 0, ws], x0s, sem),
                    (in_hbm.at[b, 1, ws], x1s, sem)]

        def out_copies(k, s):
            b, ws = unit_pos(k)
            x0s, x1s = bufs[s]
            sem = sout[s]
            return [(zb, bel_hbm.at[b, ws, 0], sem),
                    (x0s, bel_hbm.at[b, ws, 1], sem),
                    (x1s, bel_hbm.at[b, ws, 2], sem),
                    (ob, bel_hbm.at[b, ws, 3], sem)]

        def issue(copies):
            for src, dst, sem in copies:
                pltpu.async_copy(src, dst, sem)

        def drain(copies):
            for src, dst, sem in copies:
                pltpu.make_async_copy(src, dst, sem).wait()

        def stage_body(k, s):
            def body_fn():
                drain(in_copies(k, s))
                issue(out_copies(k, s))
            return body_fn

        def stage(k, s, first=False):
            if not first:
                drain(out_copies(k - 2, (s - 2) % _NSLOT))
            issue(in_copies(k + 1, (s + 1) % _NSLOT))
            stage_body(k, s)()

        issue(in_copies(0, 0))
        stage(0, 0, first=True)
        stage(1, 1, first=True)
        nbody = (MAXK - 4) // _NSLOT
        ktail = 2 + _NSLOT * nbody

        def group(g, c):
            k0 = _NSLOT * g + 2
            for j in range(_NSLOT):
                stage(k0 + j, (2 + j) % _NSLOT)
            return c

        lax.fori_loop(0, nbody, group, 0)
        for k in range(ktail, MAXK):
            s = k % _NSLOT
            if k == MAXK - 1 and full_k != NW:
                drain(out_copies(k - 2, (s - 2) % _NSLOT))
                pl.when(wid < full_k)(stage_body(k, s))
            else:
                nxt = k + 1
                if nxt < MAXK and (nxt < MAXK - 1 or full_k == NW):
                    stage(k, s)
                else:
                    drain(out_copies(k - 2, (s - 2) % _NSLOT))
                    if nxt < MAXK and nxt == MAXK - 1 and full_k != NW:
                        def issue_next(nxt=nxt, s=s):
                            issue(in_copies(nxt, (s + 1) % _NSLOT))
                        pl.when(wid < full_k)(issue_next)
                    stage_body(k, s)()
        drain(out_copies(MAXK - 2, (MAXK - 2) % _NSLOT))

        def drain_last():
            drain(out_copies(MAXK - 1, (MAXK - 1) % _NSLOT))

        if full_k == NW:
            drain_last()
        else:
            pl.when(wid < full_k)(drain_last)

    return body


@functools.lru_cache(maxsize=None)
def _build_tc_pl(B: int, W: int, H: int):
    TW = 624
    assert W % TW == 0

    def body(x_ref, o_ref):
        x0 = x_ref[0, 0]
        x1 = x_ref[0, 1]
        x2 = x_ref[0, 2]
        o_ref[0, :, 0, :] = jnp.zeros((TW, H), jnp.float32)
        o_ref[0, :, 1, :] = x0 + x2
        o_ref[0, :, 2, :] = x1 + x2
        o_ref[0, :, 3, :] = jnp.ones((TW, H), jnp.float32)

    return pl.pallas_call(
        body,
        grid=(B, W // TW),
        in_specs=[pl.BlockSpec((1, 3, TW, H), lambda b, w: (b, 0, w, 0))],
        out_specs=pl.BlockSpec((1, TW, 4, H), lambda b, w: (b, w, 0, 0)),
        out_shape=jax.ShapeDtypeStruct((B, W, 4, H), jnp.float32),
    )


def kernel(inputs):
    B, H, W, C = inputs.shape
    assert C == 3, "kernel specialized for a 2-class frame (3 input channels)"
    xt = jnp.transpose(inputs, (0, 3, 2, 1))
    bel_t = _build_sc_bel(B, W, H)(xt)
    pl_t = _build_tc_pl(B, W, H)(xt)
    bel = jnp.transpose(bel_t, (0, 3, 1, 2))
    pl_full = jnp.transpose(pl_t, (0, 3, 1, 2))
    return (bel, pl_full)
